# baseline (speedup 1.0000x reference)
import functools
import math

import jax
import jax.numpy as jnp
from jax import lax
from jax.experimental import pallas as pl
from jax.experimental.pallas import tpu as pltpu
from jax.experimental.pallas import tpu_sc as plsc

_B, _T, _D = 4, 2048, 1024
_SLOTS = (512, 256, 128)
_SALL = sum(_SLOTS)
_K = 4
_NSEL = _K * len(_SLOTS)
_LANES = 16
_N = _B * _T
_TQ = 256
_NW = 32
_QPW = _N // _NW
_WCOLS = 64
_ICOLS = _K
_S12 = _SLOTS[1] + _SLOTS[2]
_BQ = 2


def _score_topk_body(q_ref, k_ref, v12_ref, sal_ref, idx_ref, w_ref, p_ref):
    q = q_ref[...]
    k = k_ref[...]
    s = lax.dot_general(q, k, (((1,), (1,)), ((), ())),
                        preferred_element_type=jnp.float32)
    s = s * (1.0 / math.sqrt(_D)) + sal_ref[...]

    out_w = []
    out_i = []
    off = 0
    for S in _SLOTS:
        seg = s[:, off:off + S]
        col = lax.broadcasted_iota(jnp.int32, (_TQ, S), 1)
        vals, idxs = [], []
        for _ in range(_K):
            m = jnp.max(seg, axis=1, keepdims=True)
            ij = jnp.min(jnp.where(seg == m, col, S), axis=1, keepdims=True)
            vals.append(m)
            idxs.append(ij)
            seg = jnp.where(col == ij, -jnp.inf, seg)
        m0 = vals[0]
        es = [jnp.exp(v - m0) for v in vals]
        z = (es[0] + es[1]) + (es[2] + es[3])
        inv = (1.0 / len(_SLOTS)) / z
        out_w += [e * inv for e in es]
        out_i += [ij + off for ij in idxs]
        off += S

    jlane = lax.broadcasted_iota(jnp.int32, (_TQ, _WCOLS), 1) // _LANES
    wrep = jnp.zeros((_TQ, _WCOLS), jnp.float32)
    for j in range(_K):
        wrep = jnp.where(jlane == j, out_w[j], wrep)
    w_ref[...] = wrep
    idx_ref[...] = jnp.concatenate(out_i[:_K], axis=1)

    col12 = lax.broadcasted_iota(jnp.int32, (_TQ, _S12), 1) + _SLOTS[0]
    w12 = jnp.zeros((_TQ, _S12), jnp.float32)
    for j in range(_K, 3 * _K):
        w12 = jnp.where(col12 == out_i[j], out_w[j], w12)
    read12 = lax.dot_general(w12.astype(jnp.bfloat16), v12_ref[...],
                             (((1,), (0,)), ((), ())),
                             preferred_element_type=jnp.float32)
    p_ref[...] = read12.reshape(_TQ, 8, 128)


def _scores_topk(qf, k_all, v12, sal_all, interpret=False):
    return pl.pallas_call(
        _score_topk_body,
        grid=(_N // _TQ,),
        in_specs=[
            pl.BlockSpec((_TQ, _D), lambda i: (i, 0)),
            pl.BlockSpec((_SALL, _D), lambda i: (0, 0)),
            pl.BlockSpec((_S12, _D), lambda i: (0, 0)),
            pl.BlockSpec((1, _SALL), lambda i: (0, 0)),
        ],
        out_specs=[
            pl.BlockSpec((_TQ, _ICOLS), lambda i: (i, 0)),
            pl.BlockSpec((_TQ, _WCOLS), lambda i: (i, 0)),
            pl.BlockSpec((_TQ, 8, 128), lambda i: (i, 0, 0)),
        ],
        out_shape=[
            jax.ShapeDtypeStruct((_N, _ICOLS), jnp.int32),
            jax.ShapeDtypeStruct((_N, _WCOLS), jnp.float32),
            jax.ShapeDtypeStruct((_N, 8, 128), jnp.float32),
        ],
        interpret=interpret,
    )(qf, k_all, v12, sal_all)


_RING = 4


def _sc_body(v_hbm, idx_hbm, w_hbm, p_hbm, out_hbm, idx_v, w_v, rows_v, prt_v,
             row_o, *sems):
    sem_g = sems[:_RING]
    sem_p = sems[_RING:2 * _RING]
    sem_o = sems[2 * _RING:]
    wid = lax.axis_index("s") * 2 + lax.axis_index("c")
    base = wid * _QPW
    ngrp = _QPW // _BQ
    pltpu.sync_copy(
        idx_hbm.at[pl.ds(pl.multiple_of(base * _K, 8), _QPW * _K)], idx_v)
    pltpu.sync_copy(w_hbm.at[pl.ds(base, _QPW)], w_v)

    def gather(gq, par):
        go = pl.multiple_of(gq * (_BQ * _K), 8)
        return pltpu.make_async_copy(
            v_hbm.at[idx_v.at[pl.ds(go, _BQ * _K)]], rows_v.at[par],
            sem_g[par])

    def prt_copy(gq, par):
        return pltpu.make_async_copy(
            p_hbm.at[pl.ds(base + gq * _BQ, _BQ)], prt_v.at[par], sem_p[par])

    def out_copy(gq, par):
        return pltpu.make_async_copy(
            row_o.at[par], out_hbm.at[pl.ds(base + gq * _BQ, _BQ)], sem_o[par])

    for par in range(_RING):
        gather(par, par).start()
        prt_copy(par, par).start()

    def body(g, carry):
        for par in range(_RING):
            gq = g * _RING + par
            gather(gq, par).wait()
            prt_copy(gq, par).wait()

            @pl.when(g >= 1)
            def _():
                out_copy(gq - _RING, par).wait()

            for qq in range(_BQ):
                q = gq * _BQ + qq
                wbs = [w_v[q, pl.ds(j * _LANES, _LANES)] for j in range(_K)]

                def chunk(s, c2, qq=qq, wbs=wbs):
                    for l in range(8):
                        sl = pl.ds(l * _LANES, _LANES)
                        terms = [wbs[j] * rows_v[par, qq * _K + j, s, sl]
                                 for j in range(_K)]
                        terms.append(prt_v[par, qq, s, sl])
                        while len(terms) > 1:
                            terms = [terms[i] + terms[i + 1] if i + 1 < len(terms)
                                     else terms[i] for i in range(0, len(terms), 2)]
                        row_o[par, qq, s, sl] = terms[0]
                    return c2

                lax.fori_loop(0, 8, chunk, 0)

            out_copy(gq, par).start()

            @pl.when(g < ngrp // _RING - 1)
            def _():
                gather(gq + _RING, par).start()
                prt_copy(gq + _RING, par).start()
        return carry

    lax.fori_loop(0, ngrp // _RING, body, 0)
    for par in range(_RING):
        out_copy(ngrp - _RING + par, par).wait()


def _sc_gather_combine(v3, idx_flat, w, partial):
    mesh = plsc.VectorSubcoreMesh(core_axis_name="c", subcore_axis_name="s")
    run = pl.kernel(
        _sc_body,
        mesh=mesh,
        compiler_params=pltpu.CompilerParams(needs_layout_passes=False),
        out_type=jax.ShapeDtypeStruct((_N, 8, 128), jnp.float32),
        scratch_types=[
            pltpu.VMEM((_QPW * _K,), jnp.int32),
            pltpu.VMEM((_QPW, _WCOLS), jnp.float32),
            pltpu.VMEM((_RING, _BQ * _K, 8, 128), jnp.float32),
            pltpu.VMEM((_RING, _BQ, 8, 128), jnp.float32),
            pltpu.VMEM((_RING, _BQ, 8, 128), jnp.float32),
        ] + [pltpu.SemaphoreType.DMA] * (3 * _RING),
    )
    return run(v3, idx_flat, w, partial)


def kernel(q, K0, V0, sal0, K1, V1, sal1, K2, V2, sal2):
    qf = q.reshape(_N, _D)
    k_all = jnp.concatenate([K0, K1, K2], axis=0)
    v12 = jnp.concatenate([V1, V2], axis=0).astype(jnp.bfloat16)
    sal_all = jnp.concatenate([sal0, sal1, sal2]).reshape(1, _SALL)
    idx, w, partial = _scores_topk(qf, k_all, v12, sal_all)
    out = _sc_gather_combine(V0.reshape(_SLOTS[0], 8, 128),
                             idx.reshape(_N * _K), w, partial)
    return out.reshape(_B, _T, _D)


# speedup vs baseline: 1.0187x; 1.0187x over previous
import math

import jax
import jax.numpy as jnp
from jax import lax
from jax.experimental import pallas as pl
from jax.experimental.pallas import tpu as pltpu
from jax.experimental.pallas import tpu_sc as plsc

_B, _T, _D = 4, 2048, 1024
_SLOTS = (512, 256, 128)
_SALL = sum(_SLOTS)
_K = 4
_LANES = 16
_N = _B * _T
_TQ = 256
_NW = 32
_QPW = _N // _NW
_WCOLS = 64
_ICOLS = _K
_S12 = _SLOTS[1] + _SLOTS[2]
_BQ = 2


def _score_topk_body(q_ref, k_ref, v12_ref, sal_ref, idx_ref, w_ref, p_ref):
    q = q_ref[...]
    k = k_ref[...]
    s = lax.dot_general(q, k, (((1,), (1,)), ((), ())),
                        preferred_element_type=jnp.float32)
    s = s * (1.0 / math.sqrt(_D)) + sal_ref[...]

    out_w = []
    out_i = []
    off = 0
    for S in _SLOTS:
        seg = s[:, off:off + S]
        col = lax.broadcasted_iota(jnp.int32, (_TQ, S), 1)
        vals, idxs = [], []
        for _ in range(_K):
            m = jnp.max(seg, axis=1, keepdims=True)
            ij = jnp.min(jnp.where(seg == m, col, S), axis=1, keepdims=True)
            vals.append(m)
            idxs.append(ij)
            seg = jnp.where(col == ij, -jnp.inf, seg)
        m0 = vals[0]
        es = [jnp.exp(v - m0) for v in vals]
        z = (es[0] + es[1]) + (es[2] + es[3])
        inv = (1.0 / len(_SLOTS)) / z
        out_w += [e * inv for e in es]
        out_i += [ij + off for ij in idxs]
        off += S

    jlane = lax.broadcasted_iota(jnp.int32, (_TQ, _WCOLS), 1) // _LANES
    wrep = jnp.zeros((_TQ, _WCOLS), jnp.float32)
    for j in range(_K):
        wrep = jnp.where(jlane == j, out_w[j], wrep)
    w_ref[...] = wrep
    idx_ref[...] = jnp.concatenate(out_i[:_K], axis=1)

    col12 = lax.broadcasted_iota(jnp.int32, (_TQ, _S12), 1) + _SLOTS[0]
    w12 = jnp.zeros((_TQ, _S12), jnp.float32)
    for j in range(_K, 3 * _K):
        w12 = jnp.where(col12 == out_i[j], out_w[j], w12)
    read12 = lax.dot_general(w12, v12_ref[...], (((1,), (0,)), ((), ())),
                             preferred_element_type=jnp.float32)
    p_ref[...] = read12.reshape(_TQ, 8, 128)


def _scores_topk(qf, k_all, v12, sal_all, interpret=False):
    return pl.pallas_call(
        _score_topk_body,
        grid=(_N // _TQ,),
        in_specs=[
            pl.BlockSpec((_TQ, _D), lambda i: (i, 0)),
            pl.BlockSpec((_SALL, _D), lambda i: (0, 0)),
            pl.BlockSpec((_S12, _D), lambda i: (0, 0)),
            pl.BlockSpec((1, _SALL), lambda i: (0, 0)),
        ],
        out_specs=[
            pl.BlockSpec((_TQ, _ICOLS), lambda i: (i, 0)),
            pl.BlockSpec((_TQ, _WCOLS), lambda i: (i, 0)),
            pl.BlockSpec((_TQ, 8, 128), lambda i: (i, 0, 0)),
        ],
        out_shape=[
            jax.ShapeDtypeStruct((_N, _ICOLS), jnp.int32),
            jax.ShapeDtypeStruct((_N, _WCOLS), jnp.float32),
            jax.ShapeDtypeStruct((_N, 8, 128), jnp.float32),
        ],
        interpret=interpret,
    )(qf, k_all, v12, sal_all)


_RING = 4


def _sc_body(v_hbm, idx_hbm, w_hbm, p_hbm, out_hbm, idx_v, w_v, rows_v, prt_v,
             row_o, *sems):
    sem_g = sems[:_RING]
    sem_p = sems[_RING:2 * _RING]
    sem_o = sems[2 * _RING:]
    wid = lax.axis_index("s") * 2 + lax.axis_index("c")
    base = wid * _QPW
    ngrp = _QPW // _BQ
    pltpu.sync_copy(
        idx_hbm.at[pl.ds(pl.multiple_of(base * _K, 8), _QPW * _K)], idx_v)
    pltpu.sync_copy(w_hbm.at[pl.ds(base, _QPW)], w_v)

    def gather(gq, par):
        go = pl.multiple_of(gq * (_BQ * _K), 8)
        return pltpu.make_async_copy(
            v_hbm.at[idx_v.at[pl.ds(go, _BQ * _K)]], rows_v.at[par],
            sem_g[par])

    def prt_copy(gq, par):
        return pltpu.make_async_copy(
            p_hbm.at[pl.ds(base + gq * _BQ, _BQ)], prt_v.at[par], sem_p[par])

    def out_copy(gq, par):
        return pltpu.make_async_copy(
            row_o.at[par], out_hbm.at[pl.ds(base + gq * _BQ, _BQ)], sem_o[par])

    for par in range(_RING):
        gather(par, par).start()
        prt_copy(par, par).start()

    def body(g, carry):
        for par in range(_RING):
            gq = g * _RING + par
            gather(gq, par).wait()
            prt_copy(gq, par).wait()

            @pl.when(g >= 1)
            def _():
                out_copy(gq - _RING, par).wait()

            for qq in range(_BQ):
                q = gq * _BQ + qq
                wbs = [w_v[q, pl.ds(j * _LANES, _LANES)] for j in range(_K)]

                def chunk(s, c2, qq=qq, wbs=wbs):
                    for l in range(8):
                        sl = pl.ds(l * _LANES, _LANES)
                        terms = [wbs[j] * rows_v[par, qq * _K + j, s, sl]
                                 for j in range(_K)]
                        terms.append(prt_v[par, qq, s, sl])
                        while len(terms) > 1:
                            terms = [terms[i] + terms[i + 1] if i + 1 < len(terms)
                                     else terms[i] for i in range(0, len(terms), 2)]
                        row_o[par, qq, s, sl] = terms[0]
                    return c2

                lax.fori_loop(0, 8, chunk, 0)

            out_copy(gq, par).start()

            @pl.when(g < ngrp // _RING - 1)
            def _():
                gather(gq + _RING, par).start()
                prt_copy(gq + _RING, par).start()
        return carry

    lax.fori_loop(0, ngrp // _RING, body, 0)
    for par in range(_RING):
        out_copy(ngrp - _RING + par, par).wait()


def _sc_gather_combine(v3, idx_flat, w, partial):
    mesh = plsc.VectorSubcoreMesh(core_axis_name="c", subcore_axis_name="s")
    run = pl.kernel(
        _sc_body,
        mesh=mesh,
        compiler_params=pltpu.CompilerParams(needs_layout_passes=False),
        out_type=jax.ShapeDtypeStruct((_N, 8, 128), jnp.float32),
        scratch_types=[
            pltpu.VMEM((_QPW * _K,), jnp.int32),
            pltpu.VMEM((_QPW, _WCOLS), jnp.float32),
            pltpu.VMEM((_RING, _BQ * _K, 8, 128), jnp.float32),
            pltpu.VMEM((_RING, _BQ, 8, 128), jnp.float32),
            pltpu.VMEM((_RING, _BQ, 8, 128), jnp.float32),
        ] + [pltpu.SemaphoreType.DMA] * (3 * _RING),
    )
    return run(v3, idx_flat, w, partial)


def kernel(q, K0, V0, sal0, K1, V1, sal1, K2, V2, sal2):
    qf = q.reshape(_N, _D)
    k_all = jnp.concatenate([K0, K1, K2], axis=0)
    v12 = jnp.concatenate([V1, V2], axis=0)
    sal_all = jnp.concatenate([sal0, sal1, sal2]).reshape(1, _SALL)
    idx, w, partial = _scores_topk(qf, k_all, v12, sal_all)
    out = _sc_gather_combine(V0.reshape(_SLOTS[0], 8, 128),
                             idx.reshape(_N * _K), w, partial)
    return out.reshape(_B, _T, _D)
